# Initial kernel scaffold; baseline (speedup 1.0000x reference)
import jax
import jax.numpy as jnp
from jax.experimental import pallas as pl


def kernel(x, edge_index, edge_attr, batch, atom_tables, virtual_emb, lin_l_W, lin_l_b, lin_r_W, bond_tables, bn_g, bn_b, mlp_W1, mlp_b1, mlp_bn1_g, mlp_bn1_b, mlp_W2, mlp_b2, mlp_bn2_g, mlp_bn2_b):
    raise NotImplementedError("write your pallas kernel here")


# speedup vs baseline: 1.7238x; 1.7238x over previous
import functools

import jax
import jax.numpy as jnp
from jax import lax
from jax.experimental import pallas as pl
from jax.experimental.pallas import tpu as pltpu

N = 10000
E = 320000
D = 128
L = 5
G = 128


def _bn(x, g, b):
    m = x.mean(axis=0)
    v = ((x - m) ** 2).mean(axis=0)
    return (x - m) / jnp.sqrt(v + 1e-5) * g + b


def _dot_ref(a, b):
    return jnp.dot(a, b, preferred_element_type=jnp.float32)


def _split_dot3(a_bf16, b_f32, dims=None):
    hi = b_f32.astype(jnp.bfloat16)
    r1 = b_f32 - hi.astype(jnp.float32)
    lo = r1.astype(jnp.bfloat16)
    lo2 = (r1 - lo.astype(jnp.float32)).astype(jnp.bfloat16)
    if dims is None:
        dims = (((a_bf16.ndim - 1,), (0,)), ((), ()))
    acc = lax.dot_general(a_bf16, hi, dims, preferred_element_type=jnp.float32)
    acc += lax.dot_general(a_bf16, lo, dims, preferred_element_type=jnp.float32)
    acc += lax.dot_general(a_bf16, lo2, dims, preferred_element_type=jnp.float32)
    return acc


NB = 10
BLK = N // NB


def _node_mm_kernel(sum_ref, cnt_ref, hin_ref, wl_ref, bl_ref, wr_ref,
                    out_ref, s1_ref, s2_ref):
    i = pl.program_id(0)
    mean = sum_ref[...] / cnt_ref[...]
    out = (_dot_ref(mean, wl_ref[...].T) + bl_ref[...]
           + _dot_ref(hin_ref[...], wr_ref[...].T))
    out_ref[...] = out

    @pl.when(i == 0)
    def _init():
        s1_ref[...] = jnp.zeros_like(s1_ref)
        s2_ref[...] = jnp.zeros_like(s2_ref)

    s1_ref[...] += jnp.broadcast_to(
        jnp.sum(out, axis=0, keepdims=True), (8, D))
    s2_ref[...] += jnp.broadcast_to(
        jnp.sum(out * out, axis=0, keepdims=True), (8, D))


def _node_bn_kernel(last, out_ref, s1_ref, s2_ref, g_ref, b_ref, h_ref):
    m = s1_ref[0:1, :] / N
    v = s2_ref[0:1, :] / N - m * m
    y = (out_ref[...] - m) / jnp.sqrt(v + 1e-5) * g_ref[...] + b_ref[...]
    h_ref[...] = y if last else jax.nn.relu(y)


def _node_stage(last, sum_agg, cnt, hin, wl, bl, wr, bng, bnb):
    out, s1, s2 = pl.pallas_call(
        _node_mm_kernel,
        grid=(NB,),
        in_specs=[
            pl.BlockSpec((BLK, D), lambda i: (i, 0)),
            pl.BlockSpec((BLK, 1), lambda i: (i, 0)),
            pl.BlockSpec((BLK, D), lambda i: (i, 0)),
            pl.BlockSpec((D, D), lambda i: (0, 0)),
            pl.BlockSpec((1, D), lambda i: (0, 0)),
            pl.BlockSpec((D, D), lambda i: (0, 0)),
        ],
        out_specs=[
            pl.BlockSpec((BLK, D), lambda i: (i, 0)),
            pl.BlockSpec((8, D), lambda i: (0, 0)),
            pl.BlockSpec((8, D), lambda i: (0, 0)),
        ],
        out_shape=[
            jax.ShapeDtypeStruct((N, D), jnp.float32),
            jax.ShapeDtypeStruct((8, D), jnp.float32),
            jax.ShapeDtypeStruct((8, D), jnp.float32),
        ],
    )(sum_agg, cnt, hin, wl, bl.reshape(1, D), wr)
    return pl.pallas_call(
        functools.partial(_node_bn_kernel, last),
        grid=(NB,),
        in_specs=[
            pl.BlockSpec((BLK, D), lambda i: (i, 0)),
            pl.BlockSpec((8, D), lambda i: (0, 0)),
            pl.BlockSpec((8, D), lambda i: (0, 0)),
            pl.BlockSpec((1, D), lambda i: (0, 0)),
            pl.BlockSpec((1, D), lambda i: (0, 0)),
        ],
        out_specs=pl.BlockSpec((BLK, D), lambda i: (i, 0)),
        out_shape=jax.ShapeDtypeStruct((N, D), jnp.float32),
    )(out, s1, s2, bng.reshape(1, D), bnb.reshape(1, D))


def _virt_stage_kernel(hin_ref, bmat_ref, virt_ref, hnew_ref,
                       w1_ref, b1_ref, g1_ref, bb1_ref,
                       w2_ref, b2_ref, g2_ref, bb2_ref,
                       hout_ref, virtout_ref):
    bmat = bmat_ref[...]
    vt = _split_dot3(bmat, hin_ref[...], (((0,), (0,)), ((), ())))
    vt = vt + virt_ref[...]
    z = _dot_ref(vt, w1_ref[...].T) + b1_ref[...]
    z = jax.nn.relu(_bn(z, g1_ref[...], bb1_ref[...]))
    z = _dot_ref(z, w2_ref[...].T) + b2_ref[...]
    virt_new = jax.nn.relu(_bn(z, g2_ref[...], bb2_ref[...]))
    virtout_ref[...] = virt_new
    hout_ref[...] = hnew_ref[...] + _split_dot3(bmat, virt_new)


def _virt_stage(hin, bmat, virt, h_new, w1, b1, g1, bb1, w2, b2, g2, bb2):
    out_shapes = (jax.ShapeDtypeStruct((N, D), jnp.float32),
                  jax.ShapeDtypeStruct((G, D), jnp.float32))
    return pl.pallas_call(
        _virt_stage_kernel,
        out_shape=out_shapes,
    )(hin, bmat, virt, h_new, w1, b1, g1, bb1, w2, b2, g2, bb2)


def _atom_encoder_kernel(xoh_ref, tables_ref, bmat_ref, virt_ref, h_ref):
    h = _split_dot3(xoh_ref[...], tables_ref[...])
    h_ref[...] = h + _split_dot3(bmat_ref[...], virt_ref[...])


def _atom_encoder(xoh, tables, bmat, virt):
    return pl.pallas_call(
        _atom_encoder_kernel,
        out_shape=jax.ShapeDtypeStruct((N, D), jnp.float32),
    )(xoh, tables, bmat, virt)


def kernel(x, edge_index, edge_attr, batch, atom_tables, virtual_emb,
           lin_l_W, lin_l_b, lin_r_W, bond_tables, bn_g, bn_b,
           mlp_W1, mlp_b1, mlp_bn1_g, mlp_bn1_b, mlp_W2, mlp_b2,
           mlp_bn2_g, mlp_bn2_b):
    src = edge_index[0]
    dst = edge_index[1]

    xoh = (x[:, :, None] == jnp.arange(64, dtype=x.dtype)[None, None, :])
    xoh = xoh.reshape(N, 9 * 64).astype(jnp.bfloat16)
    tables_flat = atom_tables.reshape(9 * 64, D)
    bmat = (batch[:, None] == jnp.arange(G, dtype=batch.dtype)[None, :]
            ).astype(jnp.bfloat16)
    cidx = edge_attr[:, 0] * 25 + edge_attr[:, 1] * 5 + edge_attr[:, 2]
    cidx = cidx.astype(jnp.int32)
    combo = (bond_tables[:, 0][:, :, None, None, :]
             + bond_tables[:, 1][:, None, :, None, :]
             + bond_tables[:, 2][:, None, None, :, :]).reshape(L, 125, D)

    cnt = jax.ops.segment_sum(jnp.ones((E,), jnp.float32), dst, num_segments=N)
    cnt = jnp.clip(cnt, 1.0)[:, None]

    virt = jnp.broadcast_to(virtual_emb[0], (G, D))
    hin = _atom_encoder(xoh, tables_flat, bmat, virt)

    for l in range(L):
        eemb = combo[l][cidx]
        msg = jax.nn.relu(hin[src] + eemb)
        sum_agg = jax.ops.segment_sum(msg, dst, num_segments=N)
        h_new = _node_stage(l == L - 1, sum_agg, cnt, hin,
                            lin_l_W[l], lin_l_b[l], lin_r_W[l],
                            bn_g[l], bn_b[l])
        if l == L - 1:
            hin = h_new
        else:
            hin, virt = _virt_stage(hin, bmat, virt, h_new,
                                    mlp_W1[l], mlp_b1[l], mlp_bn1_g[l],
                                    mlp_bn1_b[l], mlp_W2[l], mlp_b2[l],
                                    mlp_bn2_g[l], mlp_bn2_b[l])
    return hin


# speedup vs baseline: 4.3023x; 2.4959x over previous
import functools

import jax
import jax.numpy as jnp
from jax import lax
from jax.experimental import pallas as pl
from jax.experimental.pallas import tpu as pltpu
from jax.experimental.pallas import tpu_sc as plsc

N = 10000
E = 320000
D = 128
L = 5
G = 128

CORES = 2
SUBS = 16
NW = CORES * SUBS
EPW = E // NW
CH = 80
NCH = EPW // CH
RPS = 624
TAIL_OFF = SUBS * RPS
TAIL = N - TAIL_OFF


def _sc_mesh():
    return plsc.VectorSubcoreMesh(core_axis_name="c", subcore_axis_name="s",
                                  num_cores=CORES, num_subcores=SUBS)


@functools.cache
def _edge_kernel():
    scratch = [
        pltpu.VMEM((CH,), jnp.int32),
        pltpu.VMEM((CH,), jnp.int32),
        pltpu.VMEM((CH,), jnp.int32),
        pltpu.VMEM((CH, D), jnp.float32),
        pltpu.VMEM((CH, D), jnp.float32),
        pltpu.VMEM_SHARED((N, D), jnp.float32),
        pltpu.SemaphoreType.DMA,
        pltpu.SemaphoreType.DMA,
    ]

    def body(hin_ref, src_ref, dst_ref, cid_ref, combo_ref, z_ref,
             out_ref, sidx, didx, cidx, gbuf, ebuf, acc, sem1, sem2):
        c = lax.axis_index("c")
        s = lax.axis_index("s")
        wid = c * SUBS + s
        pltpu.sync_copy(z_ref.at[pl.ds(s * RPS, RPS)],
                        acc.at[pl.ds(s * RPS, RPS)])

        @pl.when(s == SUBS - 1)
        def _tail_zero():
            pltpu.sync_copy(z_ref.at[pl.ds(TAIL_OFF, TAIL)],
                            acc.at[pl.ds(TAIL_OFF, TAIL)])

        plsc.subcore_barrier()

        def chunk(j, carry):
            pltpu.sync_copy(src_ref.at[wid, j], sidx)
            pltpu.sync_copy(dst_ref.at[wid, j], didx)
            pltpu.sync_copy(cid_ref.at[wid, j], cidx)
            pltpu.async_copy(hin_ref.at[sidx], gbuf, sem1).wait()
            pltpu.async_copy(combo_ref.at[cidx], ebuf, sem2).wait()

            def row(i, rcarry):
                for jj in range(8):
                    sl = pl.ds(jj * 16, 16)
                    v = gbuf[i, sl] + ebuf[i, sl]
                    gbuf[i, sl] = jnp.maximum(v, 0.0)
                return rcarry
            lax.fori_loop(0, CH, row, 0)
            pltpu.sync_copy(gbuf, acc.at[didx], add=True)
            return carry
        lax.fori_loop(0, NCH, chunk, 0)
        plsc.subcore_barrier()
        pltpu.sync_copy(acc.at[pl.ds(s * RPS, RPS)],
                        out_ref.at[c, pl.ds(s * RPS, RPS)])

        @pl.when(s == SUBS - 1)
        def _tail_out():
            pltpu.sync_copy(acc.at[pl.ds(TAIL_OFF, TAIL)],
                            out_ref.at[c, pl.ds(TAIL_OFF, TAIL)])

    return pl.kernel(
        body,
        [jax.ShapeDtypeStruct((CORES, N, D), jnp.float32)],
        mesh=_sc_mesh(),
        scratch_types=scratch,
    )


@functools.cache
def _cnt_kernel():
    scratch = [
        pltpu.VMEM((CH,), jnp.int32),
        pltpu.VMEM((CH, D), jnp.float32),
        pltpu.VMEM_SHARED((N, D), jnp.float32),
    ]

    def body(dst_ref, z_ref, out_ref, didx, ones, acc):
        c = lax.axis_index("c")
        s = lax.axis_index("s")
        wid = c * SUBS + s
        pltpu.sync_copy(z_ref.at[pl.ds(s * RPS, RPS)],
                        acc.at[pl.ds(s * RPS, RPS)])

        @pl.when(s == SUBS - 1)
        def _tail_zero():
            pltpu.sync_copy(z_ref.at[pl.ds(TAIL_OFF, TAIL)],
                            acc.at[pl.ds(TAIL_OFF, TAIL)])

        def fill(i, carry):
            for jj in range(8):
                ones[i, pl.ds(jj * 16, 16)] = jnp.ones((16,), jnp.float32)
            return carry
        lax.fori_loop(0, CH, fill, 0)
        plsc.subcore_barrier()

        def chunk(j, carry):
            pltpu.sync_copy(dst_ref.at[wid, j], didx)
            pltpu.sync_copy(ones, acc.at[didx], add=True)
            return carry
        lax.fori_loop(0, NCH, chunk, 0)
        plsc.subcore_barrier()
        pltpu.sync_copy(acc.at[pl.ds(s * RPS, RPS)],
                        out_ref.at[c, pl.ds(s * RPS, RPS)])

        @pl.when(s == SUBS - 1)
        def _tail_out():
            pltpu.sync_copy(acc.at[pl.ds(TAIL_OFF, TAIL)],
                            out_ref.at[c, pl.ds(TAIL_OFF, TAIL)])

    return pl.kernel(
        body,
        [jax.ShapeDtypeStruct((CORES, N, D), jnp.float32)],
        mesh=_sc_mesh(),
        scratch_types=scratch,
    )


NB = 10
BLK = N // NB


def _bn(x, g, b):
    m = x.mean(axis=0)
    v = ((x - m) ** 2).mean(axis=0)
    return (x - m) / jnp.sqrt(v + 1e-5) * g + b


def _dot_ref(a, b):
    return jnp.dot(a, b, preferred_element_type=jnp.float32)


def _split_dot3(a_bf16, b_f32, dims=None):
    hi = b_f32.astype(jnp.bfloat16)
    r1 = b_f32 - hi.astype(jnp.float32)
    lo = r1.astype(jnp.bfloat16)
    lo2 = (r1 - lo.astype(jnp.float32)).astype(jnp.bfloat16)
    if dims is None:
        dims = (((a_bf16.ndim - 1,), (0,)), ((), ()))
    acc = lax.dot_general(a_bf16, hi, dims, preferred_element_type=jnp.float32)
    acc += lax.dot_general(a_bf16, lo, dims, preferred_element_type=jnp.float32)
    acc += lax.dot_general(a_bf16, lo2, dims, preferred_element_type=jnp.float32)
    return acc


def _node_mm_kernel(p_ref, cp_ref, hin_ref, wl_ref, bl_ref, wr_ref,
                    out_ref, s1_ref, s2_ref):
    i = pl.program_id(0)
    cntv = jnp.maximum(cp_ref[0, :, 0:1] + cp_ref[1, :, 0:1], 1.0)
    mean = (p_ref[0] + p_ref[1]) / cntv
    out = (_dot_ref(mean, wl_ref[...].T) + bl_ref[...]
           + _dot_ref(hin_ref[...], wr_ref[...].T))
    out_ref[...] = out

    @pl.when(i == 0)
    def _init():
        s1_ref[...] = jnp.zeros_like(s1_ref)
        s2_ref[...] = jnp.zeros_like(s2_ref)

    s1_ref[...] += jnp.broadcast_to(
        jnp.sum(out, axis=0, keepdims=True), (8, D))
    s2_ref[...] += jnp.broadcast_to(
        jnp.sum(out * out, axis=0, keepdims=True), (8, D))


def _node_bn_kernel(last, out_ref, s1_ref, s2_ref, g_ref, b_ref, h_ref):
    m = s1_ref[0:1, :] / N
    v = s2_ref[0:1, :] / N - m * m
    y = (out_ref[...] - m) / jnp.sqrt(v + 1e-5) * g_ref[...] + b_ref[...]
    h_ref[...] = y if last else jax.nn.relu(y)


def _node_stage(last, p, cp, hin, wl, bl, wr, bng, bnb):
    out, s1, s2 = pl.pallas_call(
        _node_mm_kernel,
        grid=(NB,),
        in_specs=[
            pl.BlockSpec((2, BLK, D), lambda i: (0, i, 0)),
            pl.BlockSpec((2, BLK, D), lambda i: (0, i, 0)),
            pl.BlockSpec((BLK, D), lambda i: (i, 0)),
            pl.BlockSpec((D, D), lambda i: (0, 0)),
            pl.BlockSpec((1, D), lambda i: (0, 0)),
            pl.BlockSpec((D, D), lambda i: (0, 0)),
        ],
        out_specs=[
            pl.BlockSpec((BLK, D), lambda i: (i, 0)),
            pl.BlockSpec((8, D), lambda i: (0, 0)),
            pl.BlockSpec((8, D), lambda i: (0, 0)),
        ],
        out_shape=[
            jax.ShapeDtypeStruct((N, D), jnp.float32),
            jax.ShapeDtypeStruct((8, D), jnp.float32),
            jax.ShapeDtypeStruct((8, D), jnp.float32),
        ],
    )(p, cp, hin, wl, bl.reshape(1, D), wr)
    return pl.pallas_call(
        functools.partial(_node_bn_kernel, last),
        grid=(NB,),
        in_specs=[
            pl.BlockSpec((BLK, D), lambda i: (i, 0)),
            pl.BlockSpec((8, D), lambda i: (0, 0)),
            pl.BlockSpec((8, D), lambda i: (0, 0)),
            pl.BlockSpec((1, D), lambda i: (0, 0)),
            pl.BlockSpec((1, D), lambda i: (0, 0)),
        ],
        out_specs=pl.BlockSpec((BLK, D), lambda i: (i, 0)),
        out_shape=jax.ShapeDtypeStruct((N, D), jnp.float32),
    )(out, s1, s2, bng.reshape(1, D), bnb.reshape(1, D))


def _virt_stage_kernel(hin_ref, bmat_ref, virt_ref, hnew_ref,
                       w1_ref, b1_ref, g1_ref, bb1_ref,
                       w2_ref, b2_ref, g2_ref, bb2_ref,
                       hout_ref, virtout_ref):
    bmat = bmat_ref[...]
    vt = _split_dot3(bmat, hin_ref[...], (((0,), (0,)), ((), ())))
    vt = vt + virt_ref[...]
    z = _dot_ref(vt, w1_ref[...].T) + b1_ref[...]
    z = jax.nn.relu(_bn(z, g1_ref[...], bb1_ref[...]))
    z = _dot_ref(z, w2_ref[...].T) + b2_ref[...]
    virt_new = jax.nn.relu(_bn(z, g2_ref[...], bb2_ref[...]))
    virtout_ref[...] = virt_new
    hout_ref[...] = hnew_ref[...] + _split_dot3(bmat, virt_new)


def _virt_stage(hin, bmat, virt, h_new, w1, b1, g1, bb1, w2, b2, g2, bb2):
    out_shapes = (jax.ShapeDtypeStruct((N, D), jnp.float32),
                  jax.ShapeDtypeStruct((G, D), jnp.float32))
    return pl.pallas_call(
        _virt_stage_kernel,
        out_shape=out_shapes,
    )(hin, bmat, virt, h_new, w1, b1, g1, bb1, w2, b2, g2, bb2)


def _atom_encoder_kernel(xoh_ref, tables_ref, bmat_ref, virt_ref, h_ref):
    h = _split_dot3(xoh_ref[...], tables_ref[...])
    h_ref[...] = h + _split_dot3(bmat_ref[...], virt_ref[...])


def _atom_encoder(xoh, tables, bmat, virt):
    return pl.pallas_call(
        _atom_encoder_kernel,
        out_shape=jax.ShapeDtypeStruct((N, D), jnp.float32),
    )(xoh, tables, bmat, virt)


def kernel(x, edge_index, edge_attr, batch, atom_tables, virtual_emb,
           lin_l_W, lin_l_b, lin_r_W, bond_tables, bn_g, bn_b,
           mlp_W1, mlp_b1, mlp_bn1_g, mlp_bn1_b, mlp_W2, mlp_b2,
           mlp_bn2_g, mlp_bn2_b):
    src = edge_index[0]
    dst = edge_index[1]

    xoh = (x[:, :, None] == jnp.arange(64, dtype=x.dtype)[None, None, :])
    xoh = xoh.reshape(N, 9 * 64).astype(jnp.bfloat16)
    tables_flat = atom_tables.reshape(9 * 64, D)
    bmat = (batch[:, None] == jnp.arange(G, dtype=batch.dtype)[None, :]
            ).astype(jnp.bfloat16)
    cidx = edge_attr[:, 0] * 25 + edge_attr[:, 1] * 5 + edge_attr[:, 2]
    cidx = cidx.astype(jnp.int32)
    combo = (bond_tables[:, 0][:, :, None, None, :]
             + bond_tables[:, 1][:, None, :, None, :]
             + bond_tables[:, 2][:, None, None, :, :]).reshape(L, 125, D)

    src3 = src.astype(jnp.int32).reshape(NW, NCH, CH)
    dst3 = dst.astype(jnp.int32).reshape(NW, NCH, CH)
    cid3 = cidx.reshape(NW, NCH, CH)
    zrow = jnp.zeros((N, D), jnp.float32)

    virt = jnp.broadcast_to(virtual_emb[0], (G, D))
    hin = _atom_encoder(xoh, tables_flat, bmat, virt)

    (cp,) = _cnt_kernel()(dst3, zrow)
    for l in range(L):
        (p,) = _edge_kernel()(hin, src3, dst3, cid3, combo[l], zrow)
        h_new = _node_stage(l == L - 1, p, cp, hin,
                            lin_l_W[l], lin_l_b[l], lin_r_W[l],
                            bn_g[l], bn_b[l])
        if l == L - 1:
            hin = h_new
        else:
            hin, virt = _virt_stage(hin, bmat, virt, h_new,
                                    mlp_W1[l], mlp_b1[l], mlp_bn1_g[l],
                                    mlp_bn1_b[l], mlp_W2[l], mlp_b2[l],
                                    mlp_bn2_g[l], mlp_bn2_b[l])
    return hin
